# Initial kernel scaffold; baseline (speedup 1.0000x reference)
import jax
import jax.numpy as jnp
from jax.experimental import pallas as pl


def kernel(x, edge_index, batch, W1, b1, W2, b2, Wg, bg, Wf, bf):
    raise NotImplementedError("write your pallas kernel here")


# speedup vs baseline: 13.6760x; 13.6760x over previous
import functools

import jax
import jax.numpy as jnp
from jax import lax
from jax.experimental import pallas as pl
from jax.experimental.pallas import tpu as pltpu
from jax.experimental.pallas import tpu_sc as plsc

N = 10000
E = 320000
D = 128
HID = 256
NG = 64
NGRP = 16
NFAM = 100

NC = 2
NS = 16
NW = NC * NS
EPW = E // NW
CHUNK = 80
NCHUNK = EPW // CHUNK
RPW = 624
RTAIL = N - NS * RPW


def _zero_acc(zeros_hbm, acc, s):
    pltpu.sync_copy(zeros_hbm, acc.at[pl.ds(s * RPW, RPW)])

    @pl.when(s == NS - 1)
    def _():
        pltpu.sync_copy(zeros_hbm.at[pl.ds(0, RTAIL)],
                        acc.at[pl.ds(NS * RPW, RTAIL)])


def _write_acc(acc, out_hbm, c, s):
    pltpu.sync_copy(acc.at[pl.ds(s * RPW, RPW)],
                    out_hbm.at[c, pl.ds(s * RPW, RPW)])

    @pl.when(s == NS - 1)
    def _():
        pltpu.sync_copy(acc.at[pl.ds(NS * RPW, RTAIL)],
                        out_hbm.at[c, pl.ds(NS * RPW, RTAIL)])


def _deg_sc(dst_hbm, zeros_hbm, out_hbm, didx_v, ones_v, acc):
    c = lax.axis_index("c")
    s = lax.axis_index("s")
    wid = c * NS + s
    _zero_acc(zeros_hbm, acc, s)
    for r in range(CHUNK):
        for kk in range(8):
            ones_v[r, pl.ds(kk * 16, 16)] = jnp.ones((16,), jnp.float32)
    plsc.subcore_barrier()
    base0 = wid * EPW

    def body(i, carry):
        pltpu.sync_copy(dst_hbm.at[pl.ds(base0 + i * CHUNK, CHUNK)], didx_v)
        pltpu.sync_copy(ones_v, acc.at[didx_v], add=True)
        return carry

    lax.fori_loop(0, NCHUNK, body, 0)
    plsc.subcore_barrier()
    _write_acc(acc, out_hbm, c, s)


def _deg_call(dst):
    zeros = jnp.zeros((RPW, D), jnp.float32)
    k = pl.kernel(
        _deg_sc,
        out_type=jax.ShapeDtypeStruct((NC, N, D), jnp.float32),
        mesh=plsc.VectorSubcoreMesh(core_axis_name="c", subcore_axis_name="s", num_cores=NC, num_subcores=NS),
        scratch_types=[
            pltpu.VMEM((CHUNK,), jnp.int32),
            pltpu.VMEM((CHUNK, D), jnp.float32),
            pltpu.VMEM_SHARED((N, D), jnp.float32),
        ],
    )
    return k(dst, zeros)[:, :, :16]


def _spmm_sc(u_hbm, src_hbm, dst_hbm, zeros_hbm, out_hbm,
             sidx_v, didx_v, rows_v, acc, sem):
    c = lax.axis_index("c")
    s = lax.axis_index("s")
    wid = c * NS + s
    _zero_acc(zeros_hbm, acc, s)
    plsc.subcore_barrier()
    base0 = wid * EPW

    def body(i, carry):
        base = base0 + i * CHUNK
        pltpu.sync_copy(src_hbm.at[pl.ds(base, CHUNK)], sidx_v)
        pltpu.sync_copy(dst_hbm.at[pl.ds(base, CHUNK)], didx_v)
        pltpu.async_copy(u_hbm.at[sidx_v], rows_v, sem).wait()
        pltpu.sync_copy(rows_v, acc.at[didx_v], add=True)
        return carry

    lax.fori_loop(0, NCHUNK, body, 0)
    plsc.subcore_barrier()
    _write_acc(acc, out_hbm, c, s)


def _spmm_call(u, src, dst):
    zeros = jnp.zeros((RPW, D), jnp.float32)
    k = pl.kernel(
        _spmm_sc,
        out_type=jax.ShapeDtypeStruct((NC, N, D), jnp.float32),
        mesh=plsc.VectorSubcoreMesh(core_axis_name="c", subcore_axis_name="s", num_cores=NC, num_subcores=NS),
        scratch_types=[
            pltpu.VMEM((CHUNK,), jnp.int32),
            pltpu.VMEM((CHUNK,), jnp.int32),
            pltpu.VMEM((CHUNK, D), jnp.float32),
            pltpu.VMEM_SHARED((N, D), jnp.float32),
            pltpu.SemaphoreType.DMA,
        ],
    )
    return k(u, src, dst, zeros)


BM = 1000


def _dinv_of(dp_blk):
    deg = 1.0 + dp_blk[0, :, 0:1] + dp_blk[1, :, 0:1]
    return lax.rsqrt(deg)


def _prep_tc(x_ref, dp_ref, u0_ref):
    dinv = _dinv_of(dp_ref[...])
    u0_ref[...] = x_ref[...] * dinv


def _prep_call(x, degp):
    grid = (N // BM,)
    return pl.pallas_call(
        _prep_tc,
        grid=grid,
        in_specs=[
            pl.BlockSpec((BM, D), lambda i: (i, 0)),
            pl.BlockSpec((NC, BM, 16), lambda i: (0, i, 0)),
        ],
        out_specs=pl.BlockSpec((BM, D), lambda i: (i, 0)),
        out_shape=jax.ShapeDtypeStruct((N, D), jnp.float32),
    )(x, degp)


def _mid_tc(sp_ref, u0_ref, dp_ref, w1_ref, b1_ref, w2_ref, v1_ref):
    dinv = _dinv_of(dp_ref[...])
    t1 = dinv * (sp_ref[0] + sp_ref[1] + u0_ref[...])
    h1 = jnp.maximum(
        jnp.dot(t1, w1_ref[...], preferred_element_type=jnp.float32)
        + b1_ref[...], 0.0)
    v1_ref[...] = dinv * jnp.dot(h1, w2_ref[...],
                                 preferred_element_type=jnp.float32)


def _mid_call(s0p, u0, degp, W1, b1, W2):
    grid = (N // BM,)
    return pl.pallas_call(
        _mid_tc,
        grid=grid,
        in_specs=[
            pl.BlockSpec((NC, BM, D), lambda i: (0, i, 0)),
            pl.BlockSpec((BM, D), lambda i: (i, 0)),
            pl.BlockSpec((NC, BM, 16), lambda i: (0, i, 0)),
            pl.BlockSpec((D, HID), lambda i: (0, 0)),
            pl.BlockSpec((1, HID), lambda i: (0, 0)),
            pl.BlockSpec((HID, D), lambda i: (0, 0)),
        ],
        out_specs=pl.BlockSpec((BM, D), lambda i: (i, 0)),
        out_shape=jax.ShapeDtypeStruct((N, D), jnp.float32),
    )(s0p, u0, degp, W1, b1.reshape(1, HID), W2)


def _final_tc(sp_ref, v1_ref, dp_ref, b2_ref, seg_ref, wg_ref, bg_ref,
              wf_ref, bf_ref, pooled_ref, gl_ref, fl_ref, pacc, cacc):
    i = pl.program_id(0)
    nsteps = pl.num_programs(0)
    dinv = _dinv_of(dp_ref[...])
    h2 = dinv * (sp_ref[0] + sp_ref[1] + v1_ref[...]) + b2_ref[...]

    seg = seg_ref[0]
    gids = lax.broadcasted_iota(jnp.int32, (NG, BM), 0)
    oh = (gids == seg).astype(jnp.float32)

    @pl.when(i == 0)
    def _init():
        pacc[...] = jnp.zeros_like(pacc)
        cacc[...] = jnp.zeros_like(cacc)

    pacc[...] += jnp.dot(oh, h2, preferred_element_type=jnp.float32)
    cacc[...] += jnp.sum(oh, axis=1, keepdims=True)

    @pl.when(i == nsteps - 1)
    def _head():
        counts = jnp.maximum(cacc[...], 1.0)
        pooled = pacc[...] / counts
        gl = jnp.dot(pooled, wg_ref[...],
                     preferred_element_type=jnp.float32) + bg_ref[...]
        rowmax = jnp.max(gl, axis=1, keepdims=True)
        gidx = lax.broadcasted_iota(jnp.int32, (NG, NGRP), 1)
        pred = jnp.min(jnp.where(gl == rowmax, gidx, NGRP),
                       axis=1, keepdims=True)
        ohg = (gidx == pred).astype(jnp.float32)
        fl = jnp.zeros((NG, NFAM), jnp.float32)
        for g in range(NGRP):
            flg = jnp.dot(pooled, wf_ref[g],
                          preferred_element_type=jnp.float32) + bf_ref[g:g + 1]
            fl = fl + ohg[:, g:g + 1] * flg
        pooled_ref[...] = pooled
        gl_ref[...] = gl
        fl_ref[...] = fl


def _final_call(s1p, v1, degp, b2, batch, Wg, bg, Wf, bf):
    grid = (N // BM,)
    seg3 = batch.reshape(N // BM, 1, BM)
    return pl.pallas_call(
        _final_tc,
        grid=grid,
        in_specs=[
            pl.BlockSpec((NC, BM, D), lambda i: (0, i, 0)),
            pl.BlockSpec((BM, D), lambda i: (i, 0)),
            pl.BlockSpec((NC, BM, 16), lambda i: (0, i, 0)),
            pl.BlockSpec((1, D), lambda i: (0, 0)),
            pl.BlockSpec((1, 1, BM), lambda i: (i, 0, 0)),
            pl.BlockSpec((D, NGRP), lambda i: (0, 0)),
            pl.BlockSpec((1, NGRP), lambda i: (0, 0)),
            pl.BlockSpec((NGRP, D, NFAM), lambda i: (0, 0, 0)),
            pl.BlockSpec((NGRP, NFAM), lambda i: (0, 0)),
        ],
        out_specs=[
            pl.BlockSpec((NG, D), lambda i: (0, 0)),
            pl.BlockSpec((NG, NGRP), lambda i: (0, 0)),
            pl.BlockSpec((NG, NFAM), lambda i: (0, 0)),
        ],
        out_shape=[
            jax.ShapeDtypeStruct((NG, D), jnp.float32),
            jax.ShapeDtypeStruct((NG, NGRP), jnp.float32),
            jax.ShapeDtypeStruct((NG, NFAM), jnp.float32),
        ],
        scratch_shapes=[
            pltpu.VMEM((NG, D), jnp.float32),
            pltpu.VMEM((NG, 1), jnp.float32),
        ],
    )(s1p, v1, degp, b2.reshape(1, D), seg3, Wg, bg.reshape(1, NGRP), Wf, bf)


def kernel(x, edge_index, batch, W1, b1, W2, b2, Wg, bg, Wf, bf):
    src = edge_index[0]
    dst = edge_index[1]
    degp = _deg_call(dst)
    u0 = _prep_call(x, degp)
    s0p = _spmm_call(u0, src, dst)
    v1 = _mid_call(s0p, u0, degp, W1, b1, W2)
    s1p = _spmm_call(v1, src, dst)
    pooled, gl, fl = _final_call(s1p, v1, degp, b2, batch, Wg, bg, Wf, bf)
    return (pooled, gl, fl)


# speedup vs baseline: 25.1408x; 1.8383x over previous
import functools

import jax
import jax.numpy as jnp
from jax import lax
from jax.experimental import pallas as pl
from jax.experimental.pallas import tpu as pltpu
from jax.experimental.pallas import tpu_sc as plsc

N = 10000
E = 320000
D = 128
HID = 256
NG = 64
NGRP = 16
NFAM = 100

NC = 2
NS = 16
NW = NC * NS
EPW = E // NW
CHUNK = 80
NCHUNK = EPW // CHUNK
ECH = 128
ECHUNKS = EPW // ECH
ETAIL = EPW - ECHUNKS * ECH
RPW = 624
RTAIL = N - NS * RPW


def _zero_acc(zeros_hbm, acc, s):
    pltpu.sync_copy(zeros_hbm, acc.at[pl.ds(s * RPW, RPW)])

    @pl.when(s == NS - 1)
    def _():
        pltpu.sync_copy(zeros_hbm.at[pl.ds(0, RTAIL)],
                        acc.at[pl.ds(NS * RPW, RTAIL)])


def _write_acc(acc, out_hbm, c, s):
    pltpu.sync_copy(acc.at[pl.ds(s * RPW, RPW)],
                    out_hbm.at[c, pl.ds(s * RPW, RPW)])

    @pl.when(s == NS - 1)
    def _():
        pltpu.sync_copy(acc.at[pl.ds(NS * RPW, RTAIL)],
                        out_hbm.at[c, pl.ds(NS * RPW, RTAIL)])


def _deg_sc(dst_hbm, zeros_hbm, out_hbm, didx_v, ones_v, acc):
    c = lax.axis_index("c")
    s = lax.axis_index("s")
    wid = c * NS + s
    _zero_acc(zeros_hbm, acc, s)
    for r in range(CHUNK):
        for kk in range(8):
            ones_v[r, pl.ds(kk * 16, 16)] = jnp.ones((16,), jnp.float32)
    plsc.subcore_barrier()
    base0 = wid * EPW

    def body(i, carry):
        pltpu.sync_copy(dst_hbm.at[pl.ds(base0 + i * CHUNK, CHUNK)], didx_v)
        pltpu.sync_copy(ones_v, acc.at[didx_v], add=True)
        return carry

    lax.fori_loop(0, NCHUNK, body, 0)
    plsc.subcore_barrier()
    _write_acc(acc, out_hbm, c, s)


def _deg_call(dst):
    zeros = jnp.zeros((RPW, D), jnp.float32)
    k = pl.kernel(
        _deg_sc,
        out_type=jax.ShapeDtypeStruct((NC, N, D), jnp.float32),
        mesh=plsc.VectorSubcoreMesh(core_axis_name="c", subcore_axis_name="s", num_cores=NC, num_subcores=NS),
        scratch_types=[
            pltpu.VMEM((CHUNK,), jnp.int32),
            pltpu.VMEM((CHUNK, D), jnp.float32),
            pltpu.VMEM_SHARED((N, D), jnp.float32),
        ],
    )
    return k(dst, zeros)[:, :, :16]


def _spmm_sc(u_hbm, src3, dst3, srcT, dstT, zeros_hbm, out_hbm,
             si_a, di_a, si_b, di_b, tsidx, tdidx, rows_a, rows_b, rows_t,
             acc, gs_a, gs_b, is_a, is_b):
    c = lax.axis_index("c")
    s = lax.axis_index("s")
    wid = c * NS + s
    _zero_acc(zeros_hbm, acc, s)
    pltpu.sync_copy(srcT.at[pl.ds(wid * ETAIL, ETAIL)], tsidx)
    pltpu.sync_copy(dstT.at[pl.ds(wid * ETAIL, ETAIL)], tdidx)
    plsc.subcore_barrier()

    def idx_async(r, si, di, sem):
        pltpu.async_copy(src3.at[wid, r], si, sem)
        pltpu.async_copy(dst3.at[wid, r], di, sem)

    def idx_wait(si, di, sem):
        pltpu.make_async_copy(src3.at[wid, 0], si, sem).wait()
        pltpu.make_async_copy(dst3.at[wid, 0], di, sem).wait()

    def gat(si, buf, sem):
        pltpu.async_copy(u_hbm.at[si], buf, sem)

    def gat_wait(buf, sem):
        pltpu.make_async_copy(u_hbm.at[si_a], buf, sem).wait()

    def sca(buf, di):
        pltpu.sync_copy(buf, acc.at[di], add=True)

    pltpu.sync_copy(src3.at[wid, 0], si_a)
    pltpu.sync_copy(dst3.at[wid, 0], di_a)
    gat(si_a, rows_a, gs_a)
    idx_async(1, si_b, di_b, is_b)

    def body(j, carry):
        idx_wait(si_b, di_b, is_b)
        gat(si_b, rows_b, gs_b)
        gat_wait(rows_a, gs_a)
        sca(rows_a, di_a)
        idx_async(2 * j + 2, si_a, di_a, is_a)
        idx_wait(si_a, di_a, is_a)
        gat(si_a, rows_a, gs_a)
        gat_wait(rows_b, gs_b)
        sca(rows_b, di_b)
        idx_async(2 * j + 3, si_b, di_b, is_b)
        return carry

    lax.fori_loop(0, ECHUNKS // 2 - 1, body, 0)
    idx_wait(si_b, di_b, is_b)
    gat(si_b, rows_b, gs_b)
    gat_wait(rows_a, gs_a)
    sca(rows_a, di_a)
    gat_wait(rows_b, gs_b)
    sca(rows_b, di_b)
    pltpu.async_copy(u_hbm.at[tsidx], rows_t, gs_a).wait()
    pltpu.sync_copy(rows_t, acc.at[tdidx], add=True)

    plsc.subcore_barrier()
    _write_acc(acc, out_hbm, c, s)


def _spmm_call(u, src3, dst3, srcT, dstT):
    zeros = jnp.zeros((RPW, D), jnp.float32)
    k = pl.kernel(
        _spmm_sc,
        out_type=jax.ShapeDtypeStruct((NC, N, D), jnp.float32),
        mesh=plsc.VectorSubcoreMesh(core_axis_name="c", subcore_axis_name="s", num_cores=NC, num_subcores=NS),
        scratch_types=[
            pltpu.VMEM((ECH,), jnp.int32),
            pltpu.VMEM((ECH,), jnp.int32),
            pltpu.VMEM((ECH,), jnp.int32),
            pltpu.VMEM((ECH,), jnp.int32),
            pltpu.VMEM((ETAIL,), jnp.int32),
            pltpu.VMEM((ETAIL,), jnp.int32),
            pltpu.VMEM((ECH, D), jnp.float32),
            pltpu.VMEM((ECH, D), jnp.float32),
            pltpu.VMEM((ETAIL, D), jnp.float32),
            pltpu.VMEM_SHARED((N, D), jnp.float32),
            pltpu.SemaphoreType.DMA,
            pltpu.SemaphoreType.DMA,
            pltpu.SemaphoreType.DMA,
            pltpu.SemaphoreType.DMA,
        ],
    )
    return k(u, src3, dst3, srcT, dstT, zeros)


def _edge_split(v):
    vt = v.reshape(NW, EPW)
    main = vt[:, :ECHUNKS * ECH].reshape(NW, ECHUNKS, ECH)
    tail = vt[:, ECHUNKS * ECH:].reshape(NW * ETAIL)
    return main, tail


BM = 1000


def _dinv_of(dp_blk):
    deg = 1.0 + dp_blk[0, :, 0:1] + dp_blk[1, :, 0:1]
    return lax.rsqrt(deg)


def _prep_tc(x_ref, dp_ref, u0_ref):
    dinv = _dinv_of(dp_ref[...])
    u0_ref[...] = x_ref[...] * dinv


def _prep_call(x, degp):
    grid = (N // BM,)
    return pl.pallas_call(
        _prep_tc,
        grid=grid,
        in_specs=[
            pl.BlockSpec((BM, D), lambda i: (i, 0)),
            pl.BlockSpec((NC, BM, 16), lambda i: (0, i, 0)),
        ],
        out_specs=pl.BlockSpec((BM, D), lambda i: (i, 0)),
        out_shape=jax.ShapeDtypeStruct((N, D), jnp.float32),
    )(x, degp)


def _mid_tc(sp_ref, u0_ref, dp_ref, w1_ref, b1_ref, w2_ref, v1_ref):
    dinv = _dinv_of(dp_ref[...])
    t1 = dinv * (sp_ref[0] + sp_ref[1] + u0_ref[...])
    h1 = jnp.maximum(
        jnp.dot(t1, w1_ref[...], preferred_element_type=jnp.float32)
        + b1_ref[...], 0.0)
    v1_ref[...] = dinv * jnp.dot(h1, w2_ref[...],
                                 preferred_element_type=jnp.float32)


def _mid_call(s0p, u0, degp, W1, b1, W2):
    grid = (N // BM,)
    return pl.pallas_call(
        _mid_tc,
        grid=grid,
        in_specs=[
            pl.BlockSpec((NC, BM, D), lambda i: (0, i, 0)),
            pl.BlockSpec((BM, D), lambda i: (i, 0)),
            pl.BlockSpec((NC, BM, 16), lambda i: (0, i, 0)),
            pl.BlockSpec((D, HID), lambda i: (0, 0)),
            pl.BlockSpec((1, HID), lambda i: (0, 0)),
            pl.BlockSpec((HID, D), lambda i: (0, 0)),
        ],
        out_specs=pl.BlockSpec((BM, D), lambda i: (i, 0)),
        out_shape=jax.ShapeDtypeStruct((N, D), jnp.float32),
    )(s0p, u0, degp, W1, b1.reshape(1, HID), W2)


def _final_tc(sp_ref, v1_ref, dp_ref, b2_ref, seg_ref, wg_ref, bg_ref,
              wf_ref, bf_ref, pooled_ref, gl_ref, fl_ref, pacc, cacc):
    i = pl.program_id(0)
    nsteps = pl.num_programs(0)
    dinv = _dinv_of(dp_ref[...])
    h2 = dinv * (sp_ref[0] + sp_ref[1] + v1_ref[...]) + b2_ref[...]

    seg = seg_ref[0]
    gids = lax.broadcasted_iota(jnp.int32, (NG, BM), 0)
    oh = (gids == seg).astype(jnp.float32)

    @pl.when(i == 0)
    def _init():
        pacc[...] = jnp.zeros_like(pacc)
        cacc[...] = jnp.zeros_like(cacc)

    pacc[...] += jnp.dot(oh, h2, preferred_element_type=jnp.float32)
    cacc[...] += jnp.sum(oh, axis=1, keepdims=True)

    @pl.when(i == nsteps - 1)
    def _head():
        counts = jnp.maximum(cacc[...], 1.0)
        pooled = pacc[...] / counts
        gl = jnp.dot(pooled, wg_ref[...],
                     preferred_element_type=jnp.float32) + bg_ref[...]
        rowmax = jnp.max(gl, axis=1, keepdims=True)
        gidx = lax.broadcasted_iota(jnp.int32, (NG, NGRP), 1)
        pred = jnp.min(jnp.where(gl == rowmax, gidx, NGRP),
                       axis=1, keepdims=True)
        ohg = (gidx == pred).astype(jnp.float32)
        fl = jnp.zeros((NG, NFAM), jnp.float32)
        for g in range(NGRP):
            flg = jnp.dot(pooled, wf_ref[g],
                          preferred_element_type=jnp.float32) + bf_ref[g:g + 1]
            fl = fl + ohg[:, g:g + 1] * flg
        pooled_ref[...] = pooled
        gl_ref[...] = gl
        fl_ref[...] = fl


def _final_call(s1p, v1, degp, b2, batch, Wg, bg, Wf, bf):
    grid = (N // BM,)
    seg3 = batch.reshape(N // BM, 1, BM)
    return pl.pallas_call(
        _final_tc,
        grid=grid,
        in_specs=[
            pl.BlockSpec((NC, BM, D), lambda i: (0, i, 0)),
            pl.BlockSpec((BM, D), lambda i: (i, 0)),
            pl.BlockSpec((NC, BM, 16), lambda i: (0, i, 0)),
            pl.BlockSpec((1, D), lambda i: (0, 0)),
            pl.BlockSpec((1, 1, BM), lambda i: (i, 0, 0)),
            pl.BlockSpec((D, NGRP), lambda i: (0, 0)),
            pl.BlockSpec((1, NGRP), lambda i: (0, 0)),
            pl.BlockSpec((NGRP, D, NFAM), lambda i: (0, 0, 0)),
            pl.BlockSpec((NGRP, NFAM), lambda i: (0, 0)),
        ],
        out_specs=[
            pl.BlockSpec((NG, D), lambda i: (0, 0)),
            pl.BlockSpec((NG, NGRP), lambda i: (0, 0)),
            pl.BlockSpec((NG, NFAM), lambda i: (0, 0)),
        ],
        out_shape=[
            jax.ShapeDtypeStruct((NG, D), jnp.float32),
            jax.ShapeDtypeStruct((NG, NGRP), jnp.float32),
            jax.ShapeDtypeStruct((NG, NFAM), jnp.float32),
        ],
        scratch_shapes=[
            pltpu.VMEM((NG, D), jnp.float32),
            pltpu.VMEM((NG, 1), jnp.float32),
        ],
    )(s1p, v1, degp, b2.reshape(1, D), seg3, Wg, bg.reshape(1, NGRP), Wf, bf)


def kernel(x, edge_index, batch, W1, b1, W2, b2, Wg, bg, Wf, bf):
    src = edge_index[0]
    dst = edge_index[1]
    src3, srcT = _edge_split(src)
    dst3, dstT = _edge_split(dst)
    degp = _deg_call(dst)
    u0 = _prep_call(x, degp)
    s0p = _spmm_call(u0, src3, dst3, srcT, dstT)
    v1 = _mid_call(s0p, u0, degp, W1, b1, W2)
    s1p = _spmm_call(v1, src3, dst3, srcT, dstT)
    pooled, gl, fl = _final_call(s1p, v1, degp, b2, batch, Wg, bg, Wf, bf)
    return (pooled, gl, fl)


# speedup vs baseline: 28.4035x; 1.1298x over previous
import functools

import jax
import jax.numpy as jnp
from jax import lax
from jax.experimental import pallas as pl
from jax.experimental.pallas import tpu as pltpu
from jax.experimental.pallas import tpu_sc as plsc

N = 10000
E = 320000
D = 128
HID = 256
NG = 64
NGRP = 16
NFAM = 100

NC = 2
NS = 16
NW = NC * NS
EPW = E // NW
CHUNK = 80
NCHUNK = EPW // CHUNK
ECH = 128
ECHUNKS = EPW // ECH
ETAIL = EPW - ECHUNKS * ECH
RPW = 624
RTAIL = N - NS * RPW


def _zero_acc(zeros_hbm, acc, s):
    pltpu.sync_copy(zeros_hbm, acc.at[pl.ds(s * RPW, RPW)])

    @pl.when(s == NS - 1)
    def _():
        pltpu.sync_copy(zeros_hbm.at[pl.ds(0, RTAIL)],
                        acc.at[pl.ds(NS * RPW, RTAIL)])


def _write_acc(acc, out_hbm, c, s):
    pltpu.sync_copy(acc.at[pl.ds(s * RPW, RPW)],
                    out_hbm.at[c, pl.ds(s * RPW, RPW)])

    @pl.when(s == NS - 1)
    def _():
        pltpu.sync_copy(acc.at[pl.ds(NS * RPW, RTAIL)],
                        out_hbm.at[c, pl.ds(NS * RPW, RTAIL)])


def _deg_sc(dst3, dstT, zeros_hbm, out_hbm, di_a, di_b, tdidx, ones_v,
            onest_v, acc, is_a, is_b):
    c = lax.axis_index("c")
    s = lax.axis_index("s")
    wid = c * NS + s
    _zero_acc(zeros_hbm, acc, s)
    for r in range(ECH):
        for kk in range(8):
            ones_v[r, pl.ds(kk * 16, 16)] = jnp.ones((16,), jnp.float32)
    for r in range(ETAIL):
        for kk in range(8):
            onest_v[r, pl.ds(kk * 16, 16)] = jnp.ones((16,), jnp.float32)
    pltpu.sync_copy(dstT.at[pl.ds(wid * ETAIL, ETAIL)], tdidx)
    plsc.subcore_barrier()

    def idx_async(r, di, sem):
        pltpu.async_copy(dst3.at[wid, r], di, sem)

    def idx_wait(di, sem):
        pltpu.make_async_copy(dst3.at[wid, 0], di, sem).wait()

    def sca(di):
        pltpu.sync_copy(ones_v, acc.at[di], add=True)

    idx_async(0, di_a, is_a)
    idx_async(1, di_b, is_b)

    def body(j, carry):
        idx_wait(di_a, is_a)
        sca(di_a)
        idx_async(2 * j + 2, di_a, is_a)
        idx_wait(di_b, is_b)
        sca(di_b)
        idx_async(2 * j + 3, di_b, is_b)
        return carry

    lax.fori_loop(0, ECHUNKS // 2 - 1, body, 0)
    idx_wait(di_a, is_a)
    sca(di_a)
    idx_wait(di_b, is_b)
    sca(di_b)
    pltpu.sync_copy(onest_v, acc.at[tdidx], add=True)
    plsc.subcore_barrier()
    _write_acc(acc, out_hbm, c, s)


def _deg_call(dst3, dstT):
    zeros = jnp.zeros((RPW, D), jnp.float32)
    k = pl.kernel(
        _deg_sc,
        out_type=jax.ShapeDtypeStruct((NC, N, D), jnp.float32),
        mesh=plsc.VectorSubcoreMesh(core_axis_name="c", subcore_axis_name="s", num_cores=NC, num_subcores=NS),
        scratch_types=[
            pltpu.VMEM((ECH,), jnp.int32),
            pltpu.VMEM((ECH,), jnp.int32),
            pltpu.VMEM((ETAIL,), jnp.int32),
            pltpu.VMEM((ECH, D), jnp.float32),
            pltpu.VMEM((ETAIL, D), jnp.float32),
            pltpu.VMEM_SHARED((N, D), jnp.float32),
            pltpu.SemaphoreType.DMA,
            pltpu.SemaphoreType.DMA,
        ],
    )
    return k(dst3, dstT, zeros)[:, :, :16]


def _spmm_sc(u_hbm, src3, dst3, srcT, dstT, zeros_hbm, out_hbm,
             si_a, di_a, si_b, di_b, tsidx, tdidx, rows_a, rows_b, rows_t,
             acc, gs_a, gs_b, is_a, is_b):
    c = lax.axis_index("c")
    s = lax.axis_index("s")
    wid = c * NS + s
    _zero_acc(zeros_hbm, acc, s)
    pltpu.sync_copy(srcT.at[pl.ds(wid * ETAIL, ETAIL)], tsidx)
    pltpu.sync_copy(dstT.at[pl.ds(wid * ETAIL, ETAIL)], tdidx)
    plsc.subcore_barrier()

    def idx_async(r, si, di, sem):
        pltpu.async_copy(src3.at[wid, r], si, sem)
        pltpu.async_copy(dst3.at[wid, r], di, sem)

    def idx_wait(si, di, sem):
        pltpu.make_async_copy(src3.at[wid, 0], si, sem).wait()
        pltpu.make_async_copy(dst3.at[wid, 0], di, sem).wait()

    def gat(si, buf, sem):
        pltpu.async_copy(u_hbm.at[si], buf, sem)

    def gat_wait(buf, sem):
        pltpu.make_async_copy(u_hbm.at[si_a], buf, sem).wait()

    def sca(buf, di):
        pltpu.sync_copy(buf, acc.at[di], add=True)

    pltpu.sync_copy(src3.at[wid, 0], si_a)
    pltpu.sync_copy(dst3.at[wid, 0], di_a)
    gat(si_a, rows_a, gs_a)
    idx_async(1, si_b, di_b, is_b)

    def body(j, carry):
        idx_wait(si_b, di_b, is_b)
        gat(si_b, rows_b, gs_b)
        gat_wait(rows_a, gs_a)
        sca(rows_a, di_a)
        idx_async(2 * j + 2, si_a, di_a, is_a)
        idx_wait(si_a, di_a, is_a)
        gat(si_a, rows_a, gs_a)
        gat_wait(rows_b, gs_b)
        sca(rows_b, di_b)
        idx_async(2 * j + 3, si_b, di_b, is_b)
        return carry

    lax.fori_loop(0, ECHUNKS // 2 - 1, body, 0)
    idx_wait(si_b, di_b, is_b)
    gat(si_b, rows_b, gs_b)
    gat_wait(rows_a, gs_a)
    sca(rows_a, di_a)
    gat_wait(rows_b, gs_b)
    sca(rows_b, di_b)
    pltpu.async_copy(u_hbm.at[tsidx], rows_t, gs_a).wait()
    pltpu.sync_copy(rows_t, acc.at[tdidx], add=True)

    plsc.subcore_barrier()
    _write_acc(acc, out_hbm, c, s)


def _spmm_call(u, src3, dst3, srcT, dstT):
    zeros = jnp.zeros((RPW, D), jnp.float32)
    k = pl.kernel(
        _spmm_sc,
        out_type=jax.ShapeDtypeStruct((NC, N, D), jnp.float32),
        mesh=plsc.VectorSubcoreMesh(core_axis_name="c", subcore_axis_name="s", num_cores=NC, num_subcores=NS),
        scratch_types=[
            pltpu.VMEM((ECH,), jnp.int32),
            pltpu.VMEM((ECH,), jnp.int32),
            pltpu.VMEM((ECH,), jnp.int32),
            pltpu.VMEM((ECH,), jnp.int32),
            pltpu.VMEM((ETAIL,), jnp.int32),
            pltpu.VMEM((ETAIL,), jnp.int32),
            pltpu.VMEM((ECH, D), jnp.float32),
            pltpu.VMEM((ECH, D), jnp.float32),
            pltpu.VMEM((ETAIL, D), jnp.float32),
            pltpu.VMEM_SHARED((N, D), jnp.float32),
            pltpu.SemaphoreType.DMA,
            pltpu.SemaphoreType.DMA,
            pltpu.SemaphoreType.DMA,
            pltpu.SemaphoreType.DMA,
        ],
    )
    return k(u, src3, dst3, srcT, dstT, zeros)


def _edge_split(v):
    vt = v.reshape(NW, EPW)
    main = vt[:, :ECHUNKS * ECH].reshape(NW, ECHUNKS, ECH)
    tail = vt[:, ECHUNKS * ECH:].reshape(NW * ETAIL)
    return main, tail


BM = 1000


def _dinv_of(dp_blk):
    deg = 1.0 + dp_blk[0, :, 0:1] + dp_blk[1, :, 0:1]
    return lax.rsqrt(deg)


def _prep_tc(x_ref, dp_ref, u0_ref):
    dinv = _dinv_of(dp_ref[...])
    u0_ref[...] = x_ref[...] * dinv


def _prep_call(x, degp):
    grid = (N // BM,)
    return pl.pallas_call(
        _prep_tc,
        grid=grid,
        in_specs=[
            pl.BlockSpec((BM, D), lambda i: (i, 0)),
            pl.BlockSpec((NC, BM, 16), lambda i: (0, i, 0)),
        ],
        out_specs=pl.BlockSpec((BM, D), lambda i: (i, 0)),
        out_shape=jax.ShapeDtypeStruct((N, D), jnp.float32),
    )(x, degp)


def _mid_tc(sp_ref, u0_ref, dp_ref, w1_ref, b1_ref, w2_ref, v1_ref):
    dinv = _dinv_of(dp_ref[...])
    t1 = dinv * (sp_ref[0] + sp_ref[1] + u0_ref[...])
    h1 = jnp.maximum(
        jnp.dot(t1, w1_ref[...], preferred_element_type=jnp.float32)
        + b1_ref[...], 0.0)
    v1_ref[...] = dinv * jnp.dot(h1, w2_ref[...],
                                 preferred_element_type=jnp.float32)


def _mid_call(s0p, u0, degp, W1, b1, W2):
    grid = (N // BM,)
    return pl.pallas_call(
        _mid_tc,
        grid=grid,
        in_specs=[
            pl.BlockSpec((NC, BM, D), lambda i: (0, i, 0)),
            pl.BlockSpec((BM, D), lambda i: (i, 0)),
            pl.BlockSpec((NC, BM, 16), lambda i: (0, i, 0)),
            pl.BlockSpec((D, HID), lambda i: (0, 0)),
            pl.BlockSpec((1, HID), lambda i: (0, 0)),
            pl.BlockSpec((HID, D), lambda i: (0, 0)),
        ],
        out_specs=pl.BlockSpec((BM, D), lambda i: (i, 0)),
        out_shape=jax.ShapeDtypeStruct((N, D), jnp.float32),
    )(s0p, u0, degp, W1, b1.reshape(1, HID), W2)


def _final_tc(sp_ref, v1_ref, dp_ref, b2_ref, seg_ref, wg_ref, bg_ref,
              wf_ref, bf_ref, pooled_ref, gl_ref, fl_ref, pacc, cacc):
    i = pl.program_id(0)
    nsteps = pl.num_programs(0)
    dinv = _dinv_of(dp_ref[...])
    h2 = dinv * (sp_ref[0] + sp_ref[1] + v1_ref[...]) + b2_ref[...]

    seg = seg_ref[0]
    gids = lax.broadcasted_iota(jnp.int32, (NG, BM), 0)
    oh = (gids == seg).astype(jnp.float32)

    @pl.when(i == 0)
    def _init():
        pacc[...] = jnp.zeros_like(pacc)
        cacc[...] = jnp.zeros_like(cacc)

    pacc[...] += jnp.dot(oh, h2, preferred_element_type=jnp.float32)
    cacc[...] += jnp.sum(oh, axis=1, keepdims=True)

    @pl.when(i == nsteps - 1)
    def _head():
        counts = jnp.maximum(cacc[...], 1.0)
        pooled = pacc[...] / counts
        gl = jnp.dot(pooled, wg_ref[...],
                     preferred_element_type=jnp.float32) + bg_ref[...]
        rowmax = jnp.max(gl, axis=1, keepdims=True)
        gidx = lax.broadcasted_iota(jnp.int32, (NG, NGRP), 1)
        pred = jnp.min(jnp.where(gl == rowmax, gidx, NGRP),
                       axis=1, keepdims=True)
        ohg = (gidx == pred).astype(jnp.float32)
        fl = jnp.zeros((NG, NFAM), jnp.float32)
        for g in range(NGRP):
            flg = jnp.dot(pooled, wf_ref[g],
                          preferred_element_type=jnp.float32) + bf_ref[g:g + 1]
            fl = fl + ohg[:, g:g + 1] * flg
        pooled_ref[...] = pooled
        gl_ref[...] = gl
        fl_ref[...] = fl


def _final_call(s1p, v1, degp, b2, batch, Wg, bg, Wf, bf):
    grid = (N // BM,)
    seg3 = batch.reshape(N // BM, 1, BM)
    return pl.pallas_call(
        _final_tc,
        grid=grid,
        in_specs=[
            pl.BlockSpec((NC, BM, D), lambda i: (0, i, 0)),
            pl.BlockSpec((BM, D), lambda i: (i, 0)),
            pl.BlockSpec((NC, BM, 16), lambda i: (0, i, 0)),
            pl.BlockSpec((1, D), lambda i: (0, 0)),
            pl.BlockSpec((1, 1, BM), lambda i: (i, 0, 0)),
            pl.BlockSpec((D, NGRP), lambda i: (0, 0)),
            pl.BlockSpec((1, NGRP), lambda i: (0, 0)),
            pl.BlockSpec((NGRP, D, NFAM), lambda i: (0, 0, 0)),
            pl.BlockSpec((NGRP, NFAM), lambda i: (0, 0)),
        ],
        out_specs=[
            pl.BlockSpec((NG, D), lambda i: (0, 0)),
            pl.BlockSpec((NG, NGRP), lambda i: (0, 0)),
            pl.BlockSpec((NG, NFAM), lambda i: (0, 0)),
        ],
        out_shape=[
            jax.ShapeDtypeStruct((NG, D), jnp.float32),
            jax.ShapeDtypeStruct((NG, NGRP), jnp.float32),
            jax.ShapeDtypeStruct((NG, NFAM), jnp.float32),
        ],
        scratch_shapes=[
            pltpu.VMEM((NG, D), jnp.float32),
            pltpu.VMEM((NG, 1), jnp.float32),
        ],
    )(s1p, v1, degp, b2.reshape(1, D), seg3, Wg, bg.reshape(1, NGRP), Wf, bf)


def kernel(x, edge_index, batch, W1, b1, W2, b2, Wg, bg, Wf, bf):
    src = edge_index[0]
    dst = edge_index[1]
    src3, srcT = _edge_split(src)
    dst3, dstT = _edge_split(dst)
    degp = _deg_call(dst3, dstT)
    u0 = _prep_call(x, degp)
    s0p = _spmm_call(u0, src3, dst3, srcT, dstT)
    v1 = _mid_call(s0p, u0, degp, W1, b1, W2)
    s1p = _spmm_call(v1, src3, dst3, srcT, dstT)
    pooled, gl, fl = _final_call(s1p, v1, degp, b2, batch, Wg, bg, Wf, bf)
    return (pooled, gl, fl)
